# baseline (speedup 1.0000x reference)
import functools

import jax
import jax.numpy as jnp
from jax import lax
from jax.experimental import pallas as pl
from jax.experimental.pallas import tpu as pltpu
from jax.experimental.pallas import tpu_sc as plsc

B = 4
M = 50
SIZES = ((128, 128), (64, 64), (32, 32), (16, 16), (8, 8))
NLOC = tuple(h * w for h, w in SIZES)
NTOT = sum(NLOC)
GPB = NTOT // 16
NG = B * GPB
NWORK = 32
GPT = (NG + NWORK - 1) // NWORK
NGPAD = GPT * NWORK
BIG = 99999999.0
LANE = 16
MPAD = 64
SENT = 62
NLVL = 5
LIMS = ((-1.0, 64.0), (64.0, 128.0), (128.0, 256.0), (256.0, 512.0),
        (512.0, 999999.0))
BANDS = (32, 16, 8, 4, 1)
LPB = sum(BANDS)
GRAN = 2
NSLOT = (M + GRAN - 1) // GRAN
LSTR = 8 + NSLOT * 8 + 8
NLIST = B * LPB * LSTR


def _sc_body(gtb_hbm, jl_hbm, clso_hbm, cnto_hbm, rego_hbm,
             gtb_s, jlist_v, clsv, cntv, regv):
    wid = lax.axis_index("s") * 2 + lax.axis_index("c")
    _sc_impl(wid, gtb_hbm, jl_hbm, clso_hbm, cnto_hbm, rego_hbm,
             gtb_s, jlist_v, clsv, cntv, regv)


def _sc_impl(wid, gtb_hbm, jl_hbm, clso_hbm, cnto_hbm, rego_hbm,
             gtb_s, jlist_v, clsv, cntv, regv):
    pltpu.sync_copy(gtb_hbm, gtb_s)
    pltpu.sync_copy(jl_hbm, jlist_v)
    lane = jnp.arange(LANE, dtype=jnp.int32)
    half = jnp.full((LANE,), 0.5, jnp.float32)
    negone = jnp.full((LANE,), -1.0, jnp.float32)
    bigv = jnp.full((LANE,), BIG, jnp.float32)

    LOFFS = (0, 1024, 1280, 1344, 1360)
    NGL = (1024, 256, 64, 16, 4)
    LBO = (0, 32, 48, 56, 60)
    lo_t = wid * GPT
    hi_t = lo_t + GPT

    def _emit_slab(bb_i, lvl):
        s0 = bb_i * GPB + LOFFS[lvl]
        s1 = s0 + NGL[lvl]
        stride = 8 << lvl
        wshift = 7 - lvl
        wmask = (128 >> lvl) - 1
        rad_v = jnp.full((LANE,), float(stride), jnp.float32)
        rad2_v = jnp.full((LANE,), float(2 * stride), jnp.float32)
        lo_c, hi_c = LIMS[lvl]
        lo_v = jnp.full((LANE,), lo_c, jnp.float32)
        hi_v = jnp.full((LANE,), hi_c, jnp.float32)
        inv_s = jnp.full((LANE,), 1.0 / stride, jnp.float32)
        btab = bb_i * (MPAD * 8)
        lbase = (bb_i * LPB + LBO[lvl]) * LSTR
        bshift = 3 if lvl == 4 else 2
        group_body = _make_group_body(
            s0, stride, wshift, wmask, rad_v, rad2_v, lo_v, hi_v, inv_s,
            btab, lbase, bshift)
        glo = jnp.minimum(jnp.maximum(lo_t, s0), s1)
        ghi = jnp.maximum(jnp.minimum(hi_t, s1), glo)
        lax.fori_loop(glo, ghi, group_body, 0)

    def _make_group_body(s0, stride, wshift, wmask, rad_v, rad2_v, lo_v,
                         hi_v, inv_s, btab, lbase, bshift):
      def group_body(g, carry):
        gl = g - s0
        loc0 = gl * LANE
        locv = loc0 + lane
        ixv = locv & wmask
        iyv = lax.shift_right_logical(locv, wshift)
        xv = (ixv * stride + (stride // 2)).astype(jnp.float32)
        yv = (iyv * stride + (stride // 2)).astype(jnp.float32)
        iy0 = lax.shift_right_logical(loc0, wshift)
        band = lax.shift_right_logical(iy0, bshift)
        lb = lbase + band * LSTR

        def c_body(c, st):
            chunk = jlist_v[pl.ds(lb + 8 + c * 8, LANE)]
            for k in range(GRAN):
                besta, bl, bt, br, bb, bcls, anyp, anyi = st
                v16 = gtb_s[pl.ds(btab + chunk[k] * 8, LANE)]
                x1v = jnp.broadcast_to(v16[0], (LANE,))
                y1v = jnp.broadcast_to(v16[1], (LANE,))
                x2v = jnp.broadcast_to(v16[2], (LANE,))
                y2v = jnp.broadcast_to(v16[3], (LANE,))
                cjv = jnp.broadcast_to(v16[4], (LANE,))
                dl = xv - x1v
                dt = yv - y1v
                dr = x2v - xv
                db = y2v - yv
                omin = jnp.minimum(jnp.minimum(dl, dt), jnp.minimum(dr, db))
                omax = jnp.maximum(jnp.maximum(dl, dt), jnp.maximum(dr, db))
                cxv = (x1v + x2v) * half
                cyv = (y1v + y2v) * half
                cmx = jnp.maximum(jnp.abs(xv - cxv), jnp.abs(yv - cyv))
                e_lo = omax - lo_v
                e_hi = hi_v - omax
                m12 = jnp.minimum(omin, e_lo)
                pos_s = jnp.minimum(m12, rad_v - cmx)
                posm = jnp.where(pos_s > 0.0, e_hi, negone)
                pos = posm >= 0.0
                ign_n = jnp.minimum(jnp.minimum(e_hi, rad2_v - cmx),
                                    cmx - rad_v)
                ign = jnp.where(m12 > 0.0, ign_n, negone) >= 0.0
                area = (dl + dr) * (dt + db)
                cand = jnp.where(pos, area, bigv)
                upd = cand < besta
                besta = jnp.where(upd, cand, besta)
                bl = jnp.where(upd, dl, bl)
                bt = jnp.where(upd, dt, bt)
                br = jnp.where(upd, dr, br)
                bb = jnp.where(upd, db, bb)
                bcls = jnp.where(upd, cjv, bcls)
                one = jnp.full((LANE,), 1.0, jnp.float32)
                anyp = jnp.where(pos, one, anyp)
                anyi = jnp.where(ign, one, anyi)
                st = (besta, bl, bt, br, bb, bcls, anyp, anyi)
            return st

        zf = jnp.zeros((LANE,), jnp.float32)
        init = (jnp.full((LANE,), BIG, jnp.float32), zf, zf, zf, zf,
                zf, zf, zf)
        hdr = jlist_v[pl.ds(lb, LANE)]
        fin = lax.fori_loop(0, hdr[0], c_body, init)
        besta, bl, bt, br, bb, bclsf, anypf, anyif = fin
        bcls = bclsf.astype(jnp.int32)
        anyp = anypf > 0.5
        anyi = anyif > 0.5

        lr_min = jnp.minimum(bl, br)
        lr_max = jnp.maximum(bl, br)
        tb_min = jnp.minimum(bt, bb)
        tb_max = jnp.maximum(bt, bb)
        ratio = (lr_min * tb_min) / (lr_max * tb_max + 1e-10)
        ratio = jnp.where(anyp, ratio, 1.0)
        m = ratio
        rr = jnp.full((LANE,), 1.0, jnp.float32)
        for fac, rfac in ((4.0**16, 2.0**-16), (4.0**8, 2.0**-8),
                          (4.0**4, 2.0**-4), (4.0**2, 2.0**-2), (4.0, 0.5)):
            t = m * fac
            c = t < 1.0
            m = jnp.where(c, t, m)
            rr = jnp.where(c, rr * rfac, rr)
        y = (m + 1.0) * 0.5
        for _ in range(3):
            y = (y + m / y) * 0.5
        cnt = y * rr
        cnt = jnp.where(ratio > 1e-35, cnt, jnp.zeros((LANE,), jnp.float32))
        cnt = jnp.where(anyp, cnt, negone)
        cnt = jnp.where(anyi, negone, cnt)
        cls = jnp.where(anyp, bcls, 0)
        cls = jnp.where(anyi, -1, cls)
        o16 = (g - lo_t) * LANE
        clsv[pl.ds(o16, LANE)] = cls
        cntv[pl.ds(o16, LANE)] = cnt
        regv[pl.ds(o16, LANE)] = jnp.where(anyp, bl * inv_s, negone)
        regv[pl.ds(GPT * LANE + o16, LANE)] = jnp.where(anyp, bt * inv_s,
                                                        negone)
        regv[pl.ds(2 * GPT * LANE + o16, LANE)] = jnp.where(anyp, br * inv_s,
                                                            negone)
        regv[pl.ds(3 * GPT * LANE + o16, LANE)] = jnp.where(anyp, bb * inv_s,
                                                            negone)
        return carry

      return group_body

    for bb_i in range(B):
        for lvl in range(NLVL):
            _emit_slab(bb_i, lvl)
    pltpu.sync_copy(clsv, clso_hbm.at[pl.ds(wid * (GPT * LANE), GPT * LANE)])
    pltpu.sync_copy(cntv, cnto_hbm.at[pl.ds(wid * (GPT * LANE), GPT * LANE)])
    for c in range(4):
        pltpu.sync_copy(
            regv.at[pl.ds(c * (GPT * LANE), GPT * LANE)],
            rego_hbm.at[pl.ds(c * (NGPAD * LANE) + wid * (GPT * LANE),
                              GPT * LANE)])


def _make_sc_call(interpret=False):
    mesh = plsc.VectorSubcoreMesh(core_axis_name="c", subcore_axis_name="s",
                                  num_cores=2, num_subcores=16)
    return pl.kernel(
        _sc_body,
        out_type=(jax.ShapeDtypeStruct((NGPAD * LANE,), jnp.int32),
                  jax.ShapeDtypeStruct((NGPAD * LANE,), jnp.float32),
                  jax.ShapeDtypeStruct((NGPAD * LANE * 4,), jnp.float32)),
        mesh=mesh,
        scratch_types=[pltpu.VMEM((B * MPAD * 8,), jnp.float32),
                       pltpu.VMEM((NLIST,), jnp.int32),
                       pltpu.VMEM((GPT * LANE,), jnp.int32),
                       pltpu.VMEM((GPT * LANE,), jnp.float32),
                       pltpu.VMEM((GPT * LANE * 4,), jnp.float32)],
        interpret=interpret,
    )


_SC_CALL = _make_sc_call()


def _build_lists(gt_boxes):
    x1, y1, x2, y2 = (gt_boxes[..., i] for i in range(4))
    mx = jnp.maximum(x2 - x1, y2 - y1)
    cy = (y1 + y2) * 0.5
    jar = jnp.arange(M, dtype=jnp.int32)
    per_lvl = []
    for lvl in range(NLVL):
        s = float(8 << lvl)
        lo, hi = LIMS[lvl]
        nb = BANDS[lvl]
        rpb = (1024 // int(s)) // nb
        size_ok = ((mx > 0.0)
                   & (mx * 0.5 + (2.0 * s - lo + 0.01) > 0.0)
                   & ((hi + 0.01) - mx * 0.5 >= 0.0))
        bandi = jnp.arange(nb, dtype=jnp.float32)
        ymin = bandi * (rpb * s) + (s * 0.5)
        ymax = ymin + (rpb - 1) * s
        ok = (size_ok[:, None, :]
              & (cy[:, None, :] >= (ymin[None, :, None] - 2.0 * s - 0.01))
              & (cy[:, None, :] <= (ymax[None, :, None] + 2.0 * s + 0.01)))
        oki = ok.astype(jnp.int32)
        posi = jnp.cumsum(oki, axis=-1) - 1
        nj = posi[..., -1] + 1
        w = jnp.where(ok, (posi // GRAN) * 8 + posi % GRAN, -1)
        warr = jnp.arange(NSLOT * 8, dtype=jnp.int32)
        hit = w[..., :, None] == warr[None, None, None, :]
        vals = jnp.sum(hit * jar[None, None, :, None], axis=2)
        ent = jnp.where(jnp.any(hit, axis=2), vals, SENT).astype(jnp.int32)
        ncs = (nj + (GRAN - 1)) // GRAN
        hdr = jnp.concatenate(
            [ncs[..., None], jnp.zeros((B, nb, 7), jnp.int32)], axis=-1)
        ent = jnp.concatenate(
            [hdr, ent, jnp.full((B, nb, 8), SENT, jnp.int32)], axis=-1)
        per_lvl.append(ent)
    return jnp.concatenate(per_lvl, axis=1).reshape(-1)


def kernel(cls_logits_0, cls_logits_1, cls_logits_2, cls_logits_3,
           cls_logits_4, reg_preds_0, reg_preds_1, reg_preds_2, reg_preds_3,
           reg_preds_4, gt_boxes, classes):
    rec = jnp.concatenate(
        [gt_boxes.astype(jnp.float32),
         classes.astype(jnp.float32)[..., None],
         jnp.zeros((B, M, 3), jnp.float32)], axis=-1)
    rec = jnp.concatenate([rec, jnp.zeros((B, MPAD - M, 8), jnp.float32)],
                          axis=1)
    jl = _build_lists(gt_boxes.astype(jnp.float32))
    clso, cnto, rego = _SC_CALL(rec.reshape(-1), jl)
    cls_cat = clso[:B * NTOT].reshape(B, NTOT, 1)
    cnt_cat = cnto[:B * NTOT].reshape(B, NTOT, 1)
    reg = rego.reshape(4, NGPAD * LANE)[:, :B * NTOT]
    reg = jnp.transpose(reg.reshape(4, B, NTOT), (1, 2, 0))
    regs = []
    off = 0
    for n in NLOC:
        regs.append(reg[:, off:off + n, :])
        off += n
    return cls_cat, cnt_cat, tuple(regs)


# speedup vs baseline: 1.0091x; 1.0091x over previous
import functools

import jax
import jax.numpy as jnp
from jax import lax
from jax.experimental import pallas as pl
from jax.experimental.pallas import tpu as pltpu
from jax.experimental.pallas import tpu_sc as plsc

B = 4
M = 50
SIZES = ((128, 128), (64, 64), (32, 32), (16, 16), (8, 8))
NLOC = tuple(h * w for h, w in SIZES)
NTOT = sum(NLOC)
GPB = NTOT // 16
NG = B * GPB
NWORK = 32
GPT = (NG + NWORK - 1) // NWORK
NGPAD = GPT * NWORK
BIG = 99999999.0
LANE = 16
MPAD = 64
SENT = 62
NLVL = 5
LIMS = ((-1.0, 64.0), (64.0, 128.0), (128.0, 256.0), (256.0, 512.0),
        (512.0, 999999.0))
BANDS = (32, 16, 8, 4, 1)
LPB = sum(BANDS)
GRAN = 4
NSLOT = (M + GRAN - 1) // GRAN
LSTR = 8 + NSLOT * 8 + 8
NLIST = B * LPB * LSTR


def _sc_body(gtb_hbm, jl_hbm, clso_hbm, cnto_hbm, rego_hbm,
             gtb_s, jlist_v, clsv, cntv, regv):
    wid = lax.axis_index("s") * 2 + lax.axis_index("c")
    _sc_impl(wid, gtb_hbm, jl_hbm, clso_hbm, cnto_hbm, rego_hbm,
             gtb_s, jlist_v, clsv, cntv, regv)


def _sc_impl(wid, gtb_hbm, jl_hbm, clso_hbm, cnto_hbm, rego_hbm,
             gtb_s, jlist_v, clsv, cntv, regv):
    pltpu.sync_copy(gtb_hbm, gtb_s)
    pltpu.sync_copy(jl_hbm, jlist_v)
    lane = jnp.arange(LANE, dtype=jnp.int32)
    half = jnp.full((LANE,), 0.5, jnp.float32)
    negone = jnp.full((LANE,), -1.0, jnp.float32)
    bigv = jnp.full((LANE,), BIG, jnp.float32)

    LOFFS = (0, 1024, 1280, 1344, 1360)
    NGL = (1024, 256, 64, 16, 4)
    LBO = (0, 32, 48, 56, 60)
    lo_t = wid * GPT
    hi_t = lo_t + GPT

    def _emit_slab(bb_i, lvl):
        s0 = bb_i * GPB + LOFFS[lvl]
        s1 = s0 + NGL[lvl]
        stride = 8 << lvl
        wshift = 7 - lvl
        wmask = (128 >> lvl) - 1
        rad_v = jnp.full((LANE,), float(stride), jnp.float32)
        rad2_v = jnp.full((LANE,), float(2 * stride), jnp.float32)
        lo_c, hi_c = LIMS[lvl]
        lo_v = jnp.full((LANE,), lo_c, jnp.float32)
        hi_v = jnp.full((LANE,), hi_c, jnp.float32)
        inv_s = jnp.full((LANE,), 1.0 / stride, jnp.float32)
        btab = bb_i * (MPAD * 8)
        lbase = (bb_i * LPB + LBO[lvl]) * LSTR
        bshift = 3 if lvl == 4 else 2
        group_body = _make_group_body(
            s0, stride, wshift, wmask, rad_v, rad2_v, lo_v, hi_v, inv_s,
            btab, lbase, bshift)
        glo = jnp.minimum(jnp.maximum(lo_t, s0), s1)
        ghi = jnp.maximum(jnp.minimum(hi_t, s1), glo)
        lax.fori_loop(glo, ghi, group_body, 0)

    def _make_group_body(s0, stride, wshift, wmask, rad_v, rad2_v, lo_v,
                         hi_v, inv_s, btab, lbase, bshift):
      def group_body(g, carry):
        gl = g - s0
        loc0 = gl * LANE
        locv = loc0 + lane
        ixv = locv & wmask
        iyv = lax.shift_right_logical(locv, wshift)
        xv = (ixv * stride + (stride // 2)).astype(jnp.float32)
        yv = (iyv * stride + (stride // 2)).astype(jnp.float32)
        iy0 = lax.shift_right_logical(loc0, wshift)
        band = lax.shift_right_logical(iy0, bshift)
        lb = lbase + band * LSTR

        def c_body(c, st):
            chunk = jlist_v[pl.ds(lb + 8 + c * 8, LANE)]
            for k in range(GRAN):
                besta, bl, bt, br, bb, bcls, anyp, anyi = st
                v16 = gtb_s[pl.ds(btab + chunk[k] * 8, LANE)]
                x1v = jnp.broadcast_to(v16[0], (LANE,))
                y1v = jnp.broadcast_to(v16[1], (LANE,))
                x2v = jnp.broadcast_to(v16[2], (LANE,))
                y2v = jnp.broadcast_to(v16[3], (LANE,))
                cjv = jnp.broadcast_to(v16[4], (LANE,))
                dl = xv - x1v
                dt = yv - y1v
                dr = x2v - xv
                db = y2v - yv
                omin = jnp.minimum(jnp.minimum(dl, dt), jnp.minimum(dr, db))
                omax = jnp.maximum(jnp.maximum(dl, dt), jnp.maximum(dr, db))
                cxv = (x1v + x2v) * half
                cyv = (y1v + y2v) * half
                cmx = jnp.maximum(jnp.abs(xv - cxv), jnp.abs(yv - cyv))
                e_lo = omax - lo_v
                e_hi = hi_v - omax
                m12 = jnp.minimum(omin, e_lo)
                pos_s = jnp.minimum(m12, rad_v - cmx)
                posm = jnp.where(pos_s > 0.0, e_hi, negone)
                pos = posm >= 0.0
                ign_n = jnp.minimum(jnp.minimum(e_hi, rad2_v - cmx),
                                    cmx - rad_v)
                ign = jnp.where(m12 > 0.0, ign_n, negone) >= 0.0
                area = (dl + dr) * (dt + db)
                cand = jnp.where(pos, area, bigv)
                upd = cand < besta
                besta = jnp.where(upd, cand, besta)
                bl = jnp.where(upd, dl, bl)
                bt = jnp.where(upd, dt, bt)
                br = jnp.where(upd, dr, br)
                bb = jnp.where(upd, db, bb)
                bcls = jnp.where(upd, cjv, bcls)
                one = jnp.full((LANE,), 1.0, jnp.float32)
                anyp = jnp.where(pos, one, anyp)
                anyi = jnp.where(ign, one, anyi)
                st = (besta, bl, bt, br, bb, bcls, anyp, anyi)
            return st

        zf = jnp.zeros((LANE,), jnp.float32)
        init = (jnp.full((LANE,), BIG, jnp.float32), zf, zf, zf, zf,
                zf, zf, zf)
        hdr = jlist_v[pl.ds(lb, LANE)]
        fin = lax.fori_loop(0, hdr[0], c_body, init)
        besta, bl, bt, br, bb, bclsf, anypf, anyif = fin
        bcls = bclsf.astype(jnp.int32)
        anyp = anypf > 0.5
        anyi = anyif > 0.5

        lr_min = jnp.minimum(bl, br)
        lr_max = jnp.maximum(bl, br)
        tb_min = jnp.minimum(bt, bb)
        tb_max = jnp.maximum(bt, bb)
        ratio = (lr_min * tb_min) / (lr_max * tb_max + 1e-10)
        ratio = jnp.where(anyp, ratio, 1.0)
        m = ratio
        rr = jnp.full((LANE,), 1.0, jnp.float32)
        for fac, rfac in ((4.0**16, 2.0**-16), (4.0**8, 2.0**-8),
                          (4.0**4, 2.0**-4), (4.0**2, 2.0**-2), (4.0, 0.5)):
            t = m * fac
            c = t < 1.0
            m = jnp.where(c, t, m)
            rr = jnp.where(c, rr * rfac, rr)
        y = (m + 1.0) * 0.5
        for _ in range(3):
            y = (y + m / y) * 0.5
        cnt = y * rr
        cnt = jnp.where(ratio > 1e-35, cnt, jnp.zeros((LANE,), jnp.float32))
        cnt = jnp.where(anyp, cnt, negone)
        cnt = jnp.where(anyi, negone, cnt)
        cls = jnp.where(anyp, bcls, 0)
        cls = jnp.where(anyi, -1, cls)
        o16 = (g - lo_t) * LANE
        clsv[pl.ds(o16, LANE)] = cls
        cntv[pl.ds(o16, LANE)] = cnt
        regv[pl.ds(o16, LANE)] = jnp.where(anyp, bl * inv_s, negone)
        regv[pl.ds(GPT * LANE + o16, LANE)] = jnp.where(anyp, bt * inv_s,
                                                        negone)
        regv[pl.ds(2 * GPT * LANE + o16, LANE)] = jnp.where(anyp, br * inv_s,
                                                            negone)
        regv[pl.ds(3 * GPT * LANE + o16, LANE)] = jnp.where(anyp, bb * inv_s,
                                                            negone)
        return carry

      return group_body

    for bb_i in range(B):
        for lvl in range(NLVL):
            _emit_slab(bb_i, lvl)
    pltpu.sync_copy(clsv, clso_hbm.at[pl.ds(wid * (GPT * LANE), GPT * LANE)])
    pltpu.sync_copy(cntv, cnto_hbm.at[pl.ds(wid * (GPT * LANE), GPT * LANE)])
    for c in range(4):
        pltpu.sync_copy(
            regv.at[pl.ds(c * (GPT * LANE), GPT * LANE)],
            rego_hbm.at[pl.ds(c * (NGPAD * LANE) + wid * (GPT * LANE),
                              GPT * LANE)])


def _make_sc_call(interpret=False):
    mesh = plsc.VectorSubcoreMesh(core_axis_name="c", subcore_axis_name="s",
                                  num_cores=2, num_subcores=16)
    return pl.kernel(
        _sc_body,
        out_type=(jax.ShapeDtypeStruct((NGPAD * LANE,), jnp.int32),
                  jax.ShapeDtypeStruct((NGPAD * LANE,), jnp.float32),
                  jax.ShapeDtypeStruct((NGPAD * LANE * 4,), jnp.float32)),
        mesh=mesh,
        scratch_types=[pltpu.VMEM((B * MPAD * 8,), jnp.float32),
                       pltpu.VMEM((NLIST,), jnp.int32),
                       pltpu.VMEM((GPT * LANE,), jnp.int32),
                       pltpu.VMEM((GPT * LANE,), jnp.float32),
                       pltpu.VMEM((GPT * LANE * 4,), jnp.float32)],
        interpret=interpret,
    )


_SC_CALL = _make_sc_call()


def _build_lists(gt_boxes):
    x1, y1, x2, y2 = (gt_boxes[..., i] for i in range(4))
    mx = jnp.maximum(x2 - x1, y2 - y1)
    cy = (y1 + y2) * 0.5
    jar = jnp.arange(M, dtype=jnp.int32)
    per_lvl = []
    for lvl in range(NLVL):
        s = float(8 << lvl)
        lo, hi = LIMS[lvl]
        nb = BANDS[lvl]
        rpb = (1024 // int(s)) // nb
        size_ok = ((mx > 0.0)
                   & (mx * 0.5 + (2.0 * s - lo + 0.01) > 0.0)
                   & ((hi + 0.01) - mx * 0.5 >= 0.0))
        bandi = jnp.arange(nb, dtype=jnp.float32)
        ymin = bandi * (rpb * s) + (s * 0.5)
        ymax = ymin + (rpb - 1) * s
        ok = (size_ok[:, None, :]
              & (cy[:, None, :] >= (ymin[None, :, None] - 2.0 * s - 0.01))
              & (cy[:, None, :] <= (ymax[None, :, None] + 2.0 * s + 0.01)))
        oki = ok.astype(jnp.int32)
        posi = jnp.cumsum(oki, axis=-1) - 1
        nj = posi[..., -1] + 1
        w = jnp.where(ok, (posi // GRAN) * 8 + posi % GRAN, -1)
        warr = jnp.arange(NSLOT * 8, dtype=jnp.int32)
        hit = w[..., :, None] == warr[None, None, None, :]
        vals = jnp.sum(hit * jar[None, None, :, None], axis=2)
        ent = jnp.where(jnp.any(hit, axis=2), vals, SENT).astype(jnp.int32)
        ncs = (nj + (GRAN - 1)) // GRAN
        hdr = jnp.concatenate(
            [ncs[..., None], jnp.zeros((B, nb, 7), jnp.int32)], axis=-1)
        ent = jnp.concatenate(
            [hdr, ent, jnp.full((B, nb, 8), SENT, jnp.int32)], axis=-1)
        per_lvl.append(ent)
    return jnp.concatenate(per_lvl, axis=1).reshape(-1)


def kernel(cls_logits_0, cls_logits_1, cls_logits_2, cls_logits_3,
           cls_logits_4, reg_preds_0, reg_preds_1, reg_preds_2, reg_preds_3,
           reg_preds_4, gt_boxes, classes):
    rec = jnp.concatenate(
        [gt_boxes.astype(jnp.float32),
         classes.astype(jnp.float32)[..., None],
         jnp.zeros((B, M, 3), jnp.float32)], axis=-1)
    rec = jnp.concatenate([rec, jnp.zeros((B, MPAD - M, 8), jnp.float32)],
                          axis=1)
    jl = _build_lists(gt_boxes.astype(jnp.float32))
    clso, cnto, rego = _SC_CALL(rec.reshape(-1), jl)
    cls_cat = clso[:B * NTOT].reshape(B, NTOT, 1)
    cnt_cat = cnto[:B * NTOT].reshape(B, NTOT, 1)
    reg = rego.reshape(4, NGPAD * LANE)[:, :B * NTOT]
    reg = jnp.transpose(reg.reshape(4, B, NTOT), (1, 2, 0))
    regs = []
    off = 0
    for n in NLOC:
        regs.append(reg[:, off:off + n, :])
        off += n
    return cls_cat, cnt_cat, tuple(regs)


# speedup vs baseline: 1.0635x; 1.0540x over previous
import functools

import jax
import jax.numpy as jnp
from jax import lax
from jax.experimental import pallas as pl
from jax.experimental.pallas import tpu as pltpu
from jax.experimental.pallas import tpu_sc as plsc

B = 4
M = 50
SIZES = ((128, 128), (64, 64), (32, 32), (16, 16), (8, 8))
NLOC = tuple(h * w for h, w in SIZES)
NTOT = sum(NLOC)
GPB = NTOT // 16
NG = B * GPB
NWORK = 32
GPT = (NG + NWORK - 1) // NWORK
NGPAD = GPT * NWORK
BIG = 99999999.0
LANE = 16
MPAD = 64
SENT = 62
NLVL = 5
LIMS = ((-1.0, 64.0), (64.0, 128.0), (128.0, 256.0), (256.0, 512.0),
        (512.0, 999999.0))
BANDS = (32, 16, 8, 4, 1)
LPB = sum(BANDS)
GRAN = 4
NSLOT = (M + GRAN - 1) // GRAN
LSTR = 8 + NSLOT * 8 + 8
NLIST = B * LPB * LSTR


def _sc_body(gtb_hbm, jl_hbm, clso_hbm, cnto_hbm, rego_hbm,
             gtb_s, jlist_v, clsv, cntv, regv):
    wid = lax.axis_index("s") * 2 + lax.axis_index("c")
    _sc_impl(wid, gtb_hbm, jl_hbm, clso_hbm, cnto_hbm, rego_hbm,
             gtb_s, jlist_v, clsv, cntv, regv)


def _sc_impl(wid, gtb_hbm, jl_hbm, clso_hbm, cnto_hbm, rego_hbm,
             gtb_s, jlist_v, clsv, cntv, regv):
    pltpu.sync_copy(gtb_hbm, gtb_s)
    pltpu.sync_copy(jl_hbm, jlist_v)
    lane = jnp.arange(LANE, dtype=jnp.int32)
    half = jnp.full((LANE,), 0.5, jnp.float32)
    negone = jnp.full((LANE,), -1.0, jnp.float32)
    bigv = jnp.full((LANE,), BIG, jnp.float32)

    def group_body(gi, carry):
        g = wid * GPT + gi
        b = ((g >= GPB).astype(jnp.int32) + (g >= 2 * GPB).astype(jnp.int32)
             + (g >= 3 * GPB).astype(jnp.int32))
        r = g - b * GPB
        lvl = ((r >= 1024).astype(jnp.int32) + (r >= 1280).astype(jnp.int32)
               + (r >= 1344).astype(jnp.int32) + (r >= 1360).astype(jnp.int32))
        loff = (jnp.where(lvl >= 1, 1024, 0) + jnp.where(lvl >= 2, 256, 0)
                + jnp.where(lvl >= 3, 64, 0) + jnp.where(lvl >= 4, 16, 0))
        loc0 = (r - loff) * LANE
        strd_i = lax.shift_left(8, lvl)
        half_i = lax.shift_left(4, lvl)
        wshift = 7 - lvl
        wmask = lax.shift_right_logical(128, lvl) - 1
        locv = loc0 + lane
        ixv = locv & wmask
        iyv = lax.shift_right_logical(locv, jnp.broadcast_to(wshift, (LANE,)))
        xv = (ixv * strd_i + half_i).astype(jnp.float32)
        yv = (iyv * strd_i + half_i).astype(jnp.float32)
        rad_v = jnp.full((LANE,), strd_i, jnp.int32).astype(jnp.float32)
        rad2_v = rad_v + rad_v
        lo_i = jnp.where(lvl == 0, -1, lax.shift_left(32, lvl))
        hi_i = jnp.where(lvl == 4, 999999, lax.shift_left(64, lvl))
        lo_v = jnp.full((LANE,), lo_i, jnp.int32).astype(jnp.float32)
        hi_v = jnp.full((LANE,), hi_i, jnp.int32).astype(jnp.float32)
        inv_s = jnp.full((LANE,), 1.0, jnp.float32) / rad_v
        btab = b * (MPAD * 8)
        lvloff = (jnp.where(lvl >= 1, 32, 0) + jnp.where(lvl >= 2, 16, 0)
                  + jnp.where(lvl >= 3, 8, 0) + jnp.where(lvl >= 4, 4, 0))
        iy0 = lax.shift_right_logical(loc0, wshift)
        bshift = jnp.where(lvl == 4, 3, 2)
        band = lax.shift_right_logical(iy0, bshift)
        lb = jnp.minimum(b * (LPB * LSTR) + (lvloff + band) * LSTR,
                         NLIST - LSTR)

        def c_body(c, st):
            chunk = jlist_v[pl.ds(lb + 8 + c * 8, LANE)]
            for k in range(GRAN):
                besta, bl, bt, br, bb, bcls, anyp, anyi = st
                v16 = gtb_s[pl.ds(btab + chunk[k] * 8, LANE)]
                x1v = jnp.broadcast_to(v16[0], (LANE,))
                y1v = jnp.broadcast_to(v16[1], (LANE,))
                x2v = jnp.broadcast_to(v16[2], (LANE,))
                y2v = jnp.broadcast_to(v16[3], (LANE,))
                cjv = jnp.broadcast_to(v16[4], (LANE,))
                dl = xv - x1v
                dt = yv - y1v
                dr = x2v - xv
                db = y2v - yv
                omin = jnp.minimum(jnp.minimum(dl, dt), jnp.minimum(dr, db))
                omax = jnp.maximum(jnp.maximum(dl, dt), jnp.maximum(dr, db))
                cxv = (x1v + x2v) * half
                cyv = (y1v + y2v) * half
                cmx = jnp.maximum(jnp.abs(xv - cxv), jnp.abs(yv - cyv))
                e_lo = omax - lo_v
                e_hi = hi_v - omax
                m12 = jnp.minimum(omin, e_lo)
                pos_s = jnp.minimum(m12, rad_v - cmx)
                posm = jnp.where(pos_s > 0.0, e_hi, negone)
                pos = posm >= 0.0
                ign_n = jnp.minimum(jnp.minimum(e_hi, rad2_v - cmx),
                                    cmx - rad_v)
                ign = jnp.where(m12 > 0.0, ign_n, negone) >= 0.0
                area = (dl + dr) * (dt + db)
                cand = jnp.where(pos, area, bigv)
                upd = cand < besta
                besta = jnp.where(upd, cand, besta)
                bl = jnp.where(upd, dl, bl)
                bt = jnp.where(upd, dt, bt)
                br = jnp.where(upd, dr, br)
                bb = jnp.where(upd, db, bb)
                bcls = jnp.where(upd, cjv, bcls)
                one = jnp.full((LANE,), 1.0, jnp.float32)
                anyp = jnp.where(pos, one, anyp)
                anyi = jnp.where(ign, one, anyi)
                st = (besta, bl, bt, br, bb, bcls, anyp, anyi)
            return st

        zf = jnp.zeros((LANE,), jnp.float32)
        init = (jnp.full((LANE,), BIG, jnp.float32), zf, zf, zf, zf,
                zf, zf, zf)
        hdr = jlist_v[pl.ds(lb, LANE)]
        fin = lax.fori_loop(0, hdr[0], c_body, init)
        besta, bl, bt, br, bb, bclsf, anypf, anyif = fin
        bcls = bclsf.astype(jnp.int32)
        anyp = anypf > 0.5
        anyi = anyif > 0.5

        lr_min = jnp.minimum(bl, br)
        lr_max = jnp.maximum(bl, br)
        tb_min = jnp.minimum(bt, bb)
        tb_max = jnp.maximum(bt, bb)
        ratio = (lr_min * tb_min) / (lr_max * tb_max + 1e-10)
        ratio = jnp.where(anyp, ratio, 1.0)
        m = ratio
        rr = jnp.full((LANE,), 1.0, jnp.float32)
        for fac, rfac in ((4.0**16, 2.0**-16), (4.0**8, 2.0**-8),
                          (4.0**4, 2.0**-4), (4.0**2, 2.0**-2), (4.0, 0.5)):
            t = m * fac
            c = t < 1.0
            m = jnp.where(c, t, m)
            rr = jnp.where(c, rr * rfac, rr)
        y = (m + 1.0) * 0.5
        for _ in range(3):
            y = (y + m / y) * 0.5
        cnt = y * rr
        cnt = jnp.where(ratio > 1e-35, cnt, jnp.zeros((LANE,), jnp.float32))
        cnt = jnp.where(anyp, cnt, negone)
        cnt = jnp.where(anyi, negone, cnt)
        cls = jnp.where(anyp, bcls, 0)
        cls = jnp.where(anyi, -1, cls)
        o16 = gi * LANE
        clsv[pl.ds(o16, LANE)] = cls
        cntv[pl.ds(o16, LANE)] = cnt
        regv[pl.ds(o16, LANE)] = jnp.where(anyp, bl * inv_s, negone)
        regv[pl.ds(GPT * LANE + o16, LANE)] = jnp.where(anyp, bt * inv_s,
                                                        negone)
        regv[pl.ds(2 * GPT * LANE + o16, LANE)] = jnp.where(anyp, br * inv_s,
                                                            negone)
        regv[pl.ds(3 * GPT * LANE + o16, LANE)] = jnp.where(anyp, bb * inv_s,
                                                            negone)
        return carry

    lax.fori_loop(0, GPT, group_body, 0)
    pltpu.sync_copy(clsv, clso_hbm.at[pl.ds(wid * (GPT * LANE), GPT * LANE)])
    pltpu.sync_copy(cntv, cnto_hbm.at[pl.ds(wid * (GPT * LANE), GPT * LANE)])
    for c in range(4):
        pltpu.sync_copy(
            regv.at[pl.ds(c * (GPT * LANE), GPT * LANE)],
            rego_hbm.at[pl.ds(c * (NGPAD * LANE) + wid * (GPT * LANE),
                              GPT * LANE)])


def _make_sc_call(interpret=False):
    mesh = plsc.VectorSubcoreMesh(core_axis_name="c", subcore_axis_name="s",
                                  num_cores=2, num_subcores=16)
    return pl.kernel(
        _sc_body,
        out_type=(jax.ShapeDtypeStruct((NGPAD * LANE,), jnp.int32),
                  jax.ShapeDtypeStruct((NGPAD * LANE,), jnp.float32),
                  jax.ShapeDtypeStruct((NGPAD * LANE * 4,), jnp.float32)),
        mesh=mesh,
        scratch_types=[pltpu.VMEM((B * MPAD * 8,), jnp.float32),
                       pltpu.VMEM((NLIST,), jnp.int32),
                       pltpu.VMEM((GPT * LANE,), jnp.int32),
                       pltpu.VMEM((GPT * LANE,), jnp.float32),
                       pltpu.VMEM((GPT * LANE * 4,), jnp.float32)],
        interpret=interpret,
    )


_SC_CALL = _make_sc_call()


def _build_lists(gt_boxes):
    x1, y1, x2, y2 = (gt_boxes[..., i] for i in range(4))
    mx = jnp.maximum(x2 - x1, y2 - y1)
    cy = (y1 + y2) * 0.5
    jar = jnp.arange(M, dtype=jnp.int32)
    ok_lvl = []
    for lvl in range(NLVL):
        s = float(8 << lvl)
        lo, hi = LIMS[lvl]
        nb = BANDS[lvl]
        rpb = (1024 // int(s)) // nb
        size_ok = ((mx > 0.0)
                   & (mx * 0.5 + (2.0 * s - lo + 0.01) > 0.0)
                   & ((hi + 0.01) - mx * 0.5 >= 0.0))
        bandi = jnp.arange(nb, dtype=jnp.float32)
        ymin = bandi * (rpb * s) + (s * 0.5)
        ymax = ymin + (rpb - 1) * s
        ok_lvl.append(
            size_ok[:, None, :]
            & (cy[:, None, :] >= (ymin[None, :, None] - 2.0 * s - 0.01))
            & (cy[:, None, :] <= (ymax[None, :, None] + 2.0 * s + 0.01)))
    ok = jnp.concatenate(ok_lvl, axis=1)
    posi = jnp.cumsum(ok.astype(jnp.int32), axis=-1) - 1
    nj = posi[..., -1] + 1
    w = jnp.where(ok, (posi // GRAN) * 8 + posi % GRAN, -1)
    warr = jnp.arange(NSLOT * 8, dtype=jnp.int32)
    hit = w[..., :, None] == warr[None, None, None, :]
    vals = jnp.sum(hit * jar[None, None, :, None], axis=2)
    ent = jnp.where(jnp.any(hit, axis=2), vals, SENT).astype(jnp.int32)
    ncs = (nj + (GRAN - 1)) // GRAN
    hdr = jnp.concatenate(
        [ncs[..., None], jnp.zeros((B, LPB, 7), jnp.int32)], axis=-1)
    ent = jnp.concatenate(
        [hdr, ent, jnp.full((B, LPB, 8), SENT, jnp.int32)], axis=-1)
    return ent.reshape(-1)


def kernel(cls_logits_0, cls_logits_1, cls_logits_2, cls_logits_3,
           cls_logits_4, reg_preds_0, reg_preds_1, reg_preds_2, reg_preds_3,
           reg_preds_4, gt_boxes, classes):
    rec = jnp.concatenate(
        [gt_boxes.astype(jnp.float32),
         classes.astype(jnp.float32)[..., None],
         jnp.zeros((B, M, 3), jnp.float32)], axis=-1)
    rec = jnp.concatenate([rec, jnp.zeros((B, MPAD - M, 8), jnp.float32)],
                          axis=1)
    jl = _build_lists(gt_boxes.astype(jnp.float32))
    clso, cnto, rego = _SC_CALL(rec.reshape(-1), jl)
    cls_cat = clso[:B * NTOT].reshape(B, NTOT, 1)
    cnt_cat = cnto[:B * NTOT].reshape(B, NTOT, 1)
    reg = rego.reshape(4, NGPAD * LANE)[:, :B * NTOT]
    reg = jnp.transpose(reg.reshape(4, B, NTOT), (1, 2, 0))
    regs = []
    off = 0
    for n in NLOC:
        regs.append(reg[:, off:off + n, :])
        off += n
    return cls_cat, cnt_cat, tuple(regs)
